# Initial kernel scaffold; baseline (speedup 1.0000x reference)
import jax
import jax.numpy as jnp
from jax.experimental import pallas as pl


def kernel(feat, seed, edge_index_post, edge_index_emoji, W_post, b_post, W_emoji, b_emoji, lin0_W, lin0_b, lin1_W, lin1_b):
    raise NotImplementedError("write your pallas kernel here")


# speedup vs baseline: 7.1982x; 7.1982x over previous
import functools

import jax
import jax.numpy as jnp
from jax import lax
from jax.experimental import pallas as pl
from jax.experimental.pallas import tpu as pltpu
from jax.experimental.pallas import tpu_sc as plsc

N = 10000
NP = 10240
E = 160000
EP = 163840
ROWS = EP // 128
NW = 32
RPW = ROWS // NW
NPT = NP // 16

f32 = jnp.float32
i32 = jnp.int32

_mesh = plsc.VectorSubcoreMesh(
    core_axis_name="c", subcore_axis_name="s", num_cores=2, num_subcores=16)

_sc_params = pltpu.CompilerParams(use_tc_tiling_on_sc=False)


@functools.partial(
    pl.kernel,
    out_type=jax.ShapeDtypeStruct((2, 4, NP), f32),
    mesh=_mesh,
    compiler_params=_sc_params,
    scratch_types=[
        pltpu.VMEM((RPW, 128), i32),
        pltpu.VMEM((128,), f32),
        pltpu.VMEM_SHARED((NP,), f32),
        pltpu.VMEM_SHARED((NP,), f32),
        pltpu.VMEM_SHARED((NP,), f32),
        pltpu.VMEM_SHARED((NP,), f32),
    ])
def _sc_hist(idx_hbm, ones_hbm, zeros_hbm, out_hbm, idx_v, ones_v,
             h0, h1, h2, h3):
    c = lax.axis_index("c")
    s = lax.axis_index("s")
    w = c * 16 + s
    hists = (h0, h1, h2, h3)
    pltpu.sync_copy(ones_hbm, ones_v)
    for a in range(4):
        pltpu.sync_copy(zeros_hbm, hists[a].at[pl.ds(s * NPT, NPT)])
    plsc.subcore_barrier()
    for a in range(4):
        pltpu.sync_copy(idx_hbm.at[a].at[pl.ds(w * RPW, RPW)], idx_v)

        def body(j, carry, a=a):
            pltpu.sync_copy(ones_v, hists[a].at[idx_v.at[j]], add=True)
            return carry

        lax.fori_loop(0, RPW, body, 0)
    plsc.subcore_barrier()
    for a in range(4):
        @pl.when(s == a)
        def _(a=a):
            pltpu.sync_copy(hists[a], out_hbm.at[c].at[a])


@functools.partial(
    pl.kernel,
    out_type=(jax.ShapeDtypeStruct((2, NP, 32), f32),
              jax.ShapeDtypeStruct((2, NP, 32), f32)),
    mesh=_mesh,
    compiler_params=_sc_params,
    scratch_types=[
        pltpu.VMEM((RPW, 128), i32),
        pltpu.VMEM((RPW, 128), i32),
        pltpu.VMEM((128, 32), f32),
        pltpu.VMEM_SHARED((NP, 32), f32),
        pltpu.VMEM_SHARED((NP, 32), f32),
        pltpu.SemaphoreType.DMA,
    ])
def _sc_edges(ysp_hbm, yse_hbm, idx_hbm, zeros2_hbm, outp, oute,
              sidx, didx, rows, accp, acce, sem):
    c = lax.axis_index("c")
    s = lax.axis_index("s")
    w = c * 16 + s
    pltpu.sync_copy(zeros2_hbm, accp.at[pl.ds(s * NPT, NPT)])
    pltpu.sync_copy(zeros2_hbm, acce.at[pl.ds(s * NPT, NPT)])
    plsc.subcore_barrier()
    for ys, acc, a_src, a_dst in ((ysp_hbm, accp, 0, 1),
                                  (yse_hbm, acce, 2, 3)):
        pltpu.sync_copy(idx_hbm.at[a_src].at[pl.ds(w * RPW, RPW)], sidx)
        pltpu.sync_copy(idx_hbm.at[a_dst].at[pl.ds(w * RPW, RPW)], didx)

        def body(j, carry, ys=ys, acc=acc):
            pltpu.async_copy(ys.at[sidx.at[j]], rows, sem).wait()
            pltpu.sync_copy(rows, acc.at[didx.at[j]], add=True)
            return carry

        lax.fori_loop(0, RPW, body, 0)
    plsc.subcore_barrier()
    pltpu.sync_copy(accp.at[pl.ds(s * NPT, NPT)],
                    outp.at[c].at[pl.ds(s * NPT, NPT)])
    pltpu.sync_copy(acce.at[pl.ds(s * NPT, NPT)],
                    oute.at[c].at[pl.ds(s * NPT, NPT)])


def _tc_proj_body(featb, seedb, wc, w0, ycat_o, pf_o, ss_o):
    i = pl.program_id(0)
    y = jnp.dot(featb[...], wc[...], preferred_element_type=f32)
    y = y + seedb[...] * w0[...]
    ycat_o[...] = y
    pfp = jnp.sum(featb[...], axis=0, keepdims=True)
    ssp = jnp.sum(seedb[...]).reshape(1, 1)

    @pl.when(i == 0)
    def _():
        pf_o[...] = pfp
        ss_o[...] = ssp

    @pl.when(i > 0)
    def _():
        pf_o[...] = pf_o[...] + pfp
        ss_o[...] = ss_o[...] + ssp


_BLK = 1024


def _tc_proj(featp, seedp, wc, w0):
    return pl.pallas_call(
        _tc_proj_body,
        grid=(NP // _BLK,),
        in_specs=[
            pl.BlockSpec((_BLK, 128), lambda i: (i, 0)),
            pl.BlockSpec((_BLK, 1), lambda i: (i, 0)),
            pl.BlockSpec((128, 64), lambda i: (0, 0)),
            pl.BlockSpec((1, 64), lambda i: (0, 0)),
        ],
        out_specs=[
            pl.BlockSpec((_BLK, 64), lambda i: (i, 0)),
            pl.BlockSpec((1, 128), lambda i: (0, 0)),
            pl.BlockSpec((1, 1), lambda i: (0, 0)),
        ],
        out_shape=[
            jax.ShapeDtypeStruct((NP, 64), f32),
            jax.ShapeDtypeStruct((1, 128), f32),
            jax.ShapeDtypeStruct((1, 1), f32),
        ],
    )(featp, seedp, wc, w0)


def _tc_norm_body(ycat, degs, ysp_o, yse_o):
    dsp = degs[0, 0] + degs[1, 0]
    dse = degs[0, 2] + degs[1, 2]
    nsp = jnp.where(dsp > 0, lax.rsqrt(dsp), 0.0)[..., None]
    nse = jnp.where(dse > 0, lax.rsqrt(dse), 0.0)[..., None]
    yc = ycat[...]
    ysp_o[...] = yc[:, :, :32] * nsp
    yse_o[...] = yc[:, :, 32:] * nse


def _tc_norm(ycat3, degs4):
    return pl.pallas_call(
        _tc_norm_body,
        out_shape=[
            jax.ShapeDtypeStruct((80, 128, 32), f32),
            jax.ShapeDtypeStruct((80, 128, 32), f32),
        ],
    )(ycat3, degs4)


def _tc_final_body(partp, parte, degs, bsum, pf, ss, l0w0, l0wf, l0b,
                   l1w, l1b, out):
    aggp = partp[0] + partp[1]
    agge = parte[0] + parte[1]
    ddp = degs[0, 1] + degs[1, 1]
    dde = degs[0, 3] + degs[1, 3]
    np_ = jnp.where(ddp > 0, lax.rsqrt(ddp), 0.0)[..., None]
    ne_ = jnp.where(dde > 0, lax.rsqrt(dde), 0.0)[..., None]
    h = jnp.maximum(aggp * np_ + agge * ne_ + bsum[...], 0.0)
    row = (lax.broadcasted_iota(i32, (80, 128, 1), 0) * 128
           + lax.broadcasted_iota(i32, (80, 128, 1), 1))
    h = jnp.where(row < N, h, 0.0)
    pooled1 = h.sum(axis=1).sum(axis=0)[None, :]
    score = (ss[...] * l0w0[...]
             + jnp.dot(pf[...], l0wf[...], preferred_element_type=f32)
             + l0b[...]
             + jnp.dot(pooled1, l1w[...], preferred_element_type=f32)
             + l1b[...])
    out[...] = score


def _tc_final(partp, parte, degs4, bsum, pf, ss, l0w0, l0wf, l0b, l1w, l1b):
    return pl.pallas_call(
        _tc_final_body,
        out_shape=jax.ShapeDtypeStruct((1, 32), f32),
    )(partp, parte, degs4, bsum, pf, ss, l0w0, l0wf, l0b, l1w, l1b)


def kernel(feat, seed, edge_index_post, edge_index_emoji, W_post, b_post,
           W_emoji, b_emoji, lin0_W, lin0_b, lin1_W, lin1_b):
    featp = jnp.zeros((NP, 128), f32).at[:N].set(feat)
    seedp = jnp.zeros((NP, 1), f32).at[:N, 0].set(seed.astype(f32))
    idx = jnp.full((4, EP), NP - 1, i32)
    idx = idx.at[0, :E].set(edge_index_post[0].astype(i32))
    idx = idx.at[1, :E].set(edge_index_post[1].astype(i32))
    idx = idx.at[2, :E].set(edge_index_emoji[0].astype(i32))
    idx = idx.at[3, :E].set(edge_index_emoji[1].astype(i32))
    idx3 = idx.reshape(4, ROWS, 128)
    wc = jnp.concatenate([W_post[1:], W_emoji[1:]], axis=1)
    w0 = jnp.concatenate([W_post[0], W_emoji[0]])[None, :]
    ones128 = jnp.ones((128,), f32)
    zeros1 = jnp.zeros((NPT,), f32)
    zeros2 = jnp.zeros((NPT, 32), f32)

    ycat, pf, ss = _tc_proj(featp, seedp, wc, w0)
    degs = _sc_hist(idx3, ones128, zeros1)

    ycat3 = ycat.reshape(80, 128, 64)
    degs4 = degs.reshape(2, 4, 80, 128)
    ysp3, yse3 = _tc_norm(ycat3, degs4)

    partp, parte = _sc_edges(ysp3.reshape(NP, 32), yse3.reshape(NP, 32),
                             idx3, zeros2)

    bsum = (b_post + b_emoji).reshape(1, 1, 32)
    score = _tc_final(partp.reshape(2, 80, 128, 32),
                      parte.reshape(2, 80, 128, 32),
                      degs4, bsum, pf, ss,
                      lin0_W[0:1, :], lin0_W[1:, :], lin0_b[None, :],
                      lin1_W, lin1_b[None, :])
    return score


# speedup vs baseline: 8.1356x; 1.1302x over previous
import functools

import jax
import jax.numpy as jnp
from jax import lax
from jax.experimental import pallas as pl
from jax.experimental.pallas import tpu as pltpu
from jax.experimental.pallas import tpu_sc as plsc

N = 10000
NP = 10240
E = 160000
EP = 163840
ROWS = EP // 128
NW = 32
RPW = ROWS // NW
NPT = NP // 16

f32 = jnp.float32
i32 = jnp.int32

_mesh = plsc.VectorSubcoreMesh(
    core_axis_name="c", subcore_axis_name="s", num_cores=2, num_subcores=16)

_sc_params = pltpu.CompilerParams(use_tc_tiling_on_sc=False)


@functools.partial(
    pl.kernel,
    out_type=jax.ShapeDtypeStruct((2, 4, NP), f32),
    mesh=_mesh,
    compiler_params=_sc_params,
    scratch_types=[
        pltpu.VMEM((RPW, 128), i32),
        pltpu.VMEM((128,), f32),
        pltpu.VMEM_SHARED((NP,), f32),
        pltpu.VMEM_SHARED((NP,), f32),
        pltpu.VMEM_SHARED((NP,), f32),
        pltpu.VMEM_SHARED((NP,), f32),
        pltpu.SemaphoreType.DMA,
    ])
def _sc_hist(idx_hbm, ones_hbm, zeros_hbm, out_hbm, idx_v, ones_v,
             h0, h1, h2, h3, hsem):
    c = lax.axis_index("c")
    s = lax.axis_index("s")
    w = c * 16 + s
    hists = (h0, h1, h2, h3)
    pltpu.sync_copy(ones_hbm, ones_v)
    for a in range(4):
        pltpu.sync_copy(zeros_hbm, hists[a].at[pl.ds(s * NPT, NPT)])
    plsc.subcore_barrier()
    for a in range(4):
        pltpu.sync_copy(idx_hbm.at[a].at[pl.ds(w * RPW, RPW)], idx_v)

        def fire(j, carry, a=a):
            pltpu.async_copy(ones_v, hists[a].at[idx_v.at[j]], hsem,
                             add=True)
            return carry

        lax.fori_loop(0, RPW, fire, 0)

        def drain(j, carry, a=a):
            pltpu.make_async_copy(ones_v, hists[a].at[idx_v.at[j]],
                                  hsem).wait()
            return carry

        lax.fori_loop(0, RPW, drain, 0)
    plsc.subcore_barrier()
    for a in range(4):
        @pl.when(s == a)
        def _(a=a):
            pltpu.sync_copy(hists[a], out_hbm.at[c].at[a])


_NB = 8


@functools.partial(
    pl.kernel,
    out_type=(jax.ShapeDtypeStruct((2, NP, 32), f32),
              jax.ShapeDtypeStruct((2, NP, 32), f32)),
    mesh=_mesh,
    compiler_params=_sc_params,
    scratch_types=[
        pltpu.VMEM((RPW, 128), i32),
        pltpu.VMEM((RPW, 128), i32),
        pltpu.VMEM((_NB, 128, 32), f32),
        pltpu.VMEM_SHARED((NP, 32), f32),
        pltpu.VMEM_SHARED((NP, 32), f32),
    ] + [pltpu.SemaphoreType.DMA] * (_NB + 1))
def _sc_edges(ysp_hbm, yse_hbm, idx_hbm, zeros2_hbm, outp, oute,
              sidx, didx, rows, accp, acce, *sems):
    gsem = sems[:_NB]
    ssem = sems[_NB]
    c = lax.axis_index("c")
    s = lax.axis_index("s")
    w = c * 16 + s
    pltpu.sync_copy(zeros2_hbm, accp.at[pl.ds(s * NPT, NPT)])
    pltpu.sync_copy(zeros2_hbm, acce.at[pl.ds(s * NPT, NPT)])
    plsc.subcore_barrier()
    for ys, acc, a_src, a_dst in ((ysp_hbm, accp, 0, 1),
                                  (yse_hbm, acce, 2, 3)):
        pltpu.sync_copy(idx_hbm.at[a_src].at[pl.ds(w * RPW, RPW)], sidx)
        pltpu.sync_copy(idx_hbm.at[a_dst].at[pl.ds(w * RPW, RPW)], didx)

        def body(g, carry, ys=ys, acc=acc):
            gd = []
            for b in range(_NB):
                j = g * _NB + b
                gd.append(pltpu.async_copy(ys.at[sidx.at[j]], rows.at[b],
                                           gsem[b]))
            sd = []
            for b in range(_NB):
                j = g * _NB + b
                gd[b].wait()
                sd.append(pltpu.async_copy(rows.at[b], acc.at[didx.at[j]],
                                           ssem, add=True))
            for b in range(_NB):
                sd[b].wait()
            return carry

        lax.fori_loop(0, RPW // _NB, body, 0)
    plsc.subcore_barrier()
    pltpu.sync_copy(accp.at[pl.ds(s * NPT, NPT)],
                    outp.at[c].at[pl.ds(s * NPT, NPT)])
    pltpu.sync_copy(acce.at[pl.ds(s * NPT, NPT)],
                    oute.at[c].at[pl.ds(s * NPT, NPT)])


def _tc_proj_body(featb, seedb, wc, w0, ycat_o, pf_o, ss_o):
    i = pl.program_id(0)
    y = jnp.dot(featb[...], wc[...], preferred_element_type=f32)
    y = y + seedb[...] * w0[...]
    ycat_o[...] = y
    pfp = jnp.sum(featb[...], axis=0, keepdims=True)
    ssp = jnp.sum(seedb[...]).reshape(1, 1)

    @pl.when(i == 0)
    def _():
        pf_o[...] = pfp
        ss_o[...] = ssp

    @pl.when(i > 0)
    def _():
        pf_o[...] = pf_o[...] + pfp
        ss_o[...] = ss_o[...] + ssp


_BLK = 1024


def _tc_proj(featp, seedp, wc, w0):
    return pl.pallas_call(
        _tc_proj_body,
        grid=(NP // _BLK,),
        in_specs=[
            pl.BlockSpec((_BLK, 128), lambda i: (i, 0)),
            pl.BlockSpec((_BLK, 1), lambda i: (i, 0)),
            pl.BlockSpec((128, 64), lambda i: (0, 0)),
            pl.BlockSpec((1, 64), lambda i: (0, 0)),
        ],
        out_specs=[
            pl.BlockSpec((_BLK, 64), lambda i: (i, 0)),
            pl.BlockSpec((1, 128), lambda i: (0, 0)),
            pl.BlockSpec((1, 1), lambda i: (0, 0)),
        ],
        out_shape=[
            jax.ShapeDtypeStruct((NP, 64), f32),
            jax.ShapeDtypeStruct((1, 128), f32),
            jax.ShapeDtypeStruct((1, 1), f32),
        ],
    )(featp, seedp, wc, w0)


def _tc_norm_body(ycat, degs, ysp_o, yse_o):
    dsp = degs[0, 0] + degs[1, 0]
    dse = degs[0, 2] + degs[1, 2]
    nsp = jnp.where(dsp > 0, lax.rsqrt(dsp), 0.0)[..., None]
    nse = jnp.where(dse > 0, lax.rsqrt(dse), 0.0)[..., None]
    yc = ycat[...]
    ysp_o[...] = yc[:, :, :32] * nsp
    yse_o[...] = yc[:, :, 32:] * nse


def _tc_norm(ycat3, degs4):
    return pl.pallas_call(
        _tc_norm_body,
        out_shape=[
            jax.ShapeDtypeStruct((80, 128, 32), f32),
            jax.ShapeDtypeStruct((80, 128, 32), f32),
        ],
    )(ycat3, degs4)


def _tc_final_body(partp, parte, degs, bsum, pf, ss, l0w0, l0wf, l0b,
                   l1w, l1b, out):
    aggp = partp[0] + partp[1]
    agge = parte[0] + parte[1]
    ddp = degs[0, 1] + degs[1, 1]
    dde = degs[0, 3] + degs[1, 3]
    np_ = jnp.where(ddp > 0, lax.rsqrt(ddp), 0.0)[..., None]
    ne_ = jnp.where(dde > 0, lax.rsqrt(dde), 0.0)[..., None]
    h = jnp.maximum(aggp * np_ + agge * ne_ + bsum[...], 0.0)
    row = (lax.broadcasted_iota(i32, (80, 128, 1), 0) * 128
           + lax.broadcasted_iota(i32, (80, 128, 1), 1))
    h = jnp.where(row < N, h, 0.0)
    pooled1 = h.sum(axis=1).sum(axis=0)[None, :]
    score = (ss[...] * l0w0[...]
             + jnp.dot(pf[...], l0wf[...], preferred_element_type=f32)
             + l0b[...]
             + jnp.dot(pooled1, l1w[...], preferred_element_type=f32)
             + l1b[...])
    out[...] = score


def _tc_final(partp, parte, degs4, bsum, pf, ss, l0w0, l0wf, l0b, l1w, l1b):
    return pl.pallas_call(
        _tc_final_body,
        out_shape=jax.ShapeDtypeStruct((1, 32), f32),
    )(partp, parte, degs4, bsum, pf, ss, l0w0, l0wf, l0b, l1w, l1b)


def kernel(feat, seed, edge_index_post, edge_index_emoji, W_post, b_post,
           W_emoji, b_emoji, lin0_W, lin0_b, lin1_W, lin1_b):
    featp = jnp.zeros((NP, 128), f32).at[:N].set(feat)
    seedp = jnp.zeros((NP, 1), f32).at[:N, 0].set(seed.astype(f32))
    idx = jnp.full((4, EP), NP - 1, i32)
    idx = idx.at[0, :E].set(edge_index_post[0].astype(i32))
    idx = idx.at[1, :E].set(edge_index_post[1].astype(i32))
    idx = idx.at[2, :E].set(edge_index_emoji[0].astype(i32))
    idx = idx.at[3, :E].set(edge_index_emoji[1].astype(i32))
    idx3 = idx.reshape(4, ROWS, 128)
    wc = jnp.concatenate([W_post[1:], W_emoji[1:]], axis=1)
    w0 = jnp.concatenate([W_post[0], W_emoji[0]])[None, :]
    ones128 = jnp.ones((128,), f32)
    zeros1 = jnp.zeros((NPT,), f32)
    zeros2 = jnp.zeros((NPT, 32), f32)

    ycat, pf, ss = _tc_proj(featp, seedp, wc, w0)
    degs = _sc_hist(idx3, ones128, zeros1)

    ycat3 = ycat.reshape(80, 128, 64)
    degs4 = degs.reshape(2, 4, 80, 128)
    ysp3, yse3 = _tc_norm(ycat3, degs4)

    partp, parte = _sc_edges(ysp3.reshape(NP, 32), yse3.reshape(NP, 32),
                             idx3, zeros2)

    bsum = (b_post + b_emoji).reshape(1, 1, 32)
    score = _tc_final(partp.reshape(2, 80, 128, 32),
                      parte.reshape(2, 80, 128, 32),
                      degs4, bsum, pf, ss,
                      lin0_W[0:1, :], lin0_W[1:, :], lin0_b[None, :],
                      lin1_W, lin1_b[None, :])
    return score


# speedup vs baseline: 8.2447x; 1.0134x over previous
import functools

import jax
import jax.numpy as jnp
from jax import lax
from jax.experimental import pallas as pl
from jax.experimental.pallas import tpu as pltpu
from jax.experimental.pallas import tpu_sc as plsc

N = 10000
NP = 10240
E = 160000
EP = 163840
ROWS = EP // 128
NW = 32
RPW = ROWS // NW
NPT = NP // 16

f32 = jnp.float32
i32 = jnp.int32

_mesh = plsc.VectorSubcoreMesh(
    core_axis_name="c", subcore_axis_name="s", num_cores=2, num_subcores=16)

_sc_params = pltpu.CompilerParams(use_tc_tiling_on_sc=False)


@functools.partial(
    pl.kernel,
    out_type=jax.ShapeDtypeStruct((2, 4, NP), f32),
    mesh=_mesh,
    compiler_params=_sc_params,
    scratch_types=[
        pltpu.VMEM((RPW, 128), i32),
        pltpu.VMEM((128,), f32),
        pltpu.VMEM_SHARED((NP,), f32),
        pltpu.VMEM_SHARED((NP,), f32),
        pltpu.VMEM_SHARED((NP,), f32),
        pltpu.VMEM_SHARED((NP,), f32),
        pltpu.SemaphoreType.DMA,
    ])
def _sc_hist(idx_hbm, ones_hbm, zeros_hbm, out_hbm, idx_v, ones_v,
             h0, h1, h2, h3, hsem):
    c = lax.axis_index("c")
    s = lax.axis_index("s")
    w = c * 16 + s
    hists = (h0, h1, h2, h3)
    pltpu.sync_copy(ones_hbm, ones_v)
    for a in range(4):
        pltpu.sync_copy(zeros_hbm, hists[a].at[pl.ds(s * NPT, NPT)])
    plsc.subcore_barrier()
    for a in range(4):
        pltpu.sync_copy(idx_hbm.at[a].at[pl.ds(w * RPW, RPW)], idx_v)

        def fire(j, carry, a=a):
            pltpu.async_copy(ones_v, hists[a].at[idx_v.at[j]], hsem,
                             add=True)
            return carry

        lax.fori_loop(0, RPW, fire, 0)

        def drain(j, carry, a=a):
            pltpu.make_async_copy(ones_v, hists[a].at[idx_v.at[j]],
                                  hsem).wait()
            return carry

        lax.fori_loop(0, RPW, drain, 0)
    plsc.subcore_barrier()
    for a in range(4):
        @pl.when(s == a)
        def _(a=a):
            pltpu.sync_copy(hists[a], out_hbm.at[c].at[a])


_NB = 8


@functools.partial(
    pl.kernel,
    out_type=(jax.ShapeDtypeStruct((2, NP, 32), f32),
              jax.ShapeDtypeStruct((2, NP, 32), f32)),
    mesh=_mesh,
    compiler_params=_sc_params,
    scratch_types=[
        pltpu.VMEM((RPW, 128), i32),
        pltpu.VMEM((RPW, 128), i32),
        pltpu.VMEM((_NB, 128, 32), f32),
        pltpu.VMEM_SHARED((NP, 32), f32),
        pltpu.VMEM_SHARED((NP, 32), f32),
    ] + [pltpu.SemaphoreType.DMA] * (2 * _NB))
def _sc_edges(ysp_hbm, yse_hbm, idx_hbm, zeros2_hbm, outp, oute,
              sidx, didx, rows, accp, acce, *sems):
    gsem = sems[:_NB]
    ssem = sems[_NB:]
    c = lax.axis_index("c")
    s = lax.axis_index("s")
    w = c * 16 + s
    pltpu.sync_copy(zeros2_hbm, accp.at[pl.ds(s * NPT, NPT)])
    pltpu.sync_copy(zeros2_hbm, acce.at[pl.ds(s * NPT, NPT)])
    plsc.subcore_barrier()
    for ys, acc, a_src, a_dst in ((ysp_hbm, accp, 0, 1),
                                  (yse_hbm, acce, 2, 3)):
        pltpu.sync_copy(idx_hbm.at[a_src].at[pl.ds(w * RPW, RPW)], sidx)
        pltpu.sync_copy(idx_hbm.at[a_dst].at[pl.ds(w * RPW, RPW)], didx)

        def body(g, carry, ys=ys, acc=acc):
            gd = []
            for b in range(_NB):
                j = g * _NB + b

                @pl.when(g > 0)
                def _(b=b, j=j, acc=acc):
                    pltpu.make_async_copy(
                        rows.at[b], acc.at[didx.at[j - _NB]],
                        ssem[b]).wait()

                gd.append(pltpu.async_copy(ys.at[sidx.at[j]], rows.at[b],
                                           gsem[b]))
            for b in range(_NB):
                j = g * _NB + b
                gd[b].wait()
                pltpu.async_copy(rows.at[b], acc.at[didx.at[j]],
                                 ssem[b], add=True)
            return carry

        lax.fori_loop(0, RPW // _NB, body, 0)
        for b in range(_NB):
            pltpu.make_async_copy(rows.at[b],
                                  acc.at[didx.at[RPW - _NB + b]],
                                  ssem[b]).wait()
    plsc.subcore_barrier()
    pltpu.sync_copy(accp.at[pl.ds(s * NPT, NPT)],
                    outp.at[c].at[pl.ds(s * NPT, NPT)])
    pltpu.sync_copy(acce.at[pl.ds(s * NPT, NPT)],
                    oute.at[c].at[pl.ds(s * NPT, NPT)])


def _tc_proj_body(featb, seedb, wc, w0, ycat_o, pf_o, ss_o):
    i = pl.program_id(0)
    y = jnp.dot(featb[...], wc[...], preferred_element_type=f32)
    y = y + seedb[...] * w0[...]
    ycat_o[...] = y
    pfp = jnp.sum(featb[...], axis=0, keepdims=True)
    ssp = jnp.sum(seedb[...]).reshape(1, 1)

    @pl.when(i == 0)
    def _():
        pf_o[...] = pfp
        ss_o[...] = ssp

    @pl.when(i > 0)
    def _():
        pf_o[...] = pf_o[...] + pfp
        ss_o[...] = ss_o[...] + ssp


_BLK = 1024


def _tc_proj(featp, seedp, wc, w0):
    return pl.pallas_call(
        _tc_proj_body,
        grid=(NP // _BLK,),
        in_specs=[
            pl.BlockSpec((_BLK, 128), lambda i: (i, 0)),
            pl.BlockSpec((_BLK, 1), lambda i: (i, 0)),
            pl.BlockSpec((128, 64), lambda i: (0, 0)),
            pl.BlockSpec((1, 64), lambda i: (0, 0)),
        ],
        out_specs=[
            pl.BlockSpec((_BLK, 64), lambda i: (i, 0)),
            pl.BlockSpec((1, 128), lambda i: (0, 0)),
            pl.BlockSpec((1, 1), lambda i: (0, 0)),
        ],
        out_shape=[
            jax.ShapeDtypeStruct((NP, 64), f32),
            jax.ShapeDtypeStruct((1, 128), f32),
            jax.ShapeDtypeStruct((1, 1), f32),
        ],
    )(featp, seedp, wc, w0)


def _tc_norm_body(ycat, degs, ysp_o, yse_o):
    dsp = degs[0, 0] + degs[1, 0]
    dse = degs[0, 2] + degs[1, 2]
    nsp = jnp.where(dsp > 0, lax.rsqrt(dsp), 0.0)[..., None]
    nse = jnp.where(dse > 0, lax.rsqrt(dse), 0.0)[..., None]
    yc = ycat[...]
    ysp_o[...] = yc[:, :, :32] * nsp
    yse_o[...] = yc[:, :, 32:] * nse


def _tc_norm(ycat3, degs4):
    return pl.pallas_call(
        _tc_norm_body,
        out_shape=[
            jax.ShapeDtypeStruct((80, 128, 32), f32),
            jax.ShapeDtypeStruct((80, 128, 32), f32),
        ],
    )(ycat3, degs4)


def _tc_final_body(partp, parte, degs, bsum, pf, ss, l0w0, l0wf, l0b,
                   l1w, l1b, out):
    aggp = partp[0] + partp[1]
    agge = parte[0] + parte[1]
    ddp = degs[0, 1] + degs[1, 1]
    dde = degs[0, 3] + degs[1, 3]
    np_ = jnp.where(ddp > 0, lax.rsqrt(ddp), 0.0)[..., None]
    ne_ = jnp.where(dde > 0, lax.rsqrt(dde), 0.0)[..., None]
    h = jnp.maximum(aggp * np_ + agge * ne_ + bsum[...], 0.0)
    row = (lax.broadcasted_iota(i32, (80, 128, 1), 0) * 128
           + lax.broadcasted_iota(i32, (80, 128, 1), 1))
    h = jnp.where(row < N, h, 0.0)
    pooled1 = h.sum(axis=1).sum(axis=0)[None, :]
    score = (ss[...] * l0w0[...]
             + jnp.dot(pf[...], l0wf[...], preferred_element_type=f32)
             + l0b[...]
             + jnp.dot(pooled1, l1w[...], preferred_element_type=f32)
             + l1b[...])
    out[...] = score


def _tc_final(partp, parte, degs4, bsum, pf, ss, l0w0, l0wf, l0b, l1w, l1b):
    return pl.pallas_call(
        _tc_final_body,
        out_shape=jax.ShapeDtypeStruct((1, 32), f32),
    )(partp, parte, degs4, bsum, pf, ss, l0w0, l0wf, l0b, l1w, l1b)


def kernel(feat, seed, edge_index_post, edge_index_emoji, W_post, b_post,
           W_emoji, b_emoji, lin0_W, lin0_b, lin1_W, lin1_b):
    featp = jnp.zeros((NP, 128), f32).at[:N].set(feat)
    seedp = jnp.zeros((NP, 1), f32).at[:N, 0].set(seed.astype(f32))
    idx = jnp.full((4, EP), NP - 1, i32)
    idx = idx.at[0, :E].set(edge_index_post[0].astype(i32))
    idx = idx.at[1, :E].set(edge_index_post[1].astype(i32))
    idx = idx.at[2, :E].set(edge_index_emoji[0].astype(i32))
    idx = idx.at[3, :E].set(edge_index_emoji[1].astype(i32))
    idx3 = idx.reshape(4, ROWS, 128)
    wc = jnp.concatenate([W_post[1:], W_emoji[1:]], axis=1)
    w0 = jnp.concatenate([W_post[0], W_emoji[0]])[None, :]
    ones128 = jnp.ones((128,), f32)
    zeros1 = jnp.zeros((NPT,), f32)
    zeros2 = jnp.zeros((NPT, 32), f32)

    ycat, pf, ss = _tc_proj(featp, seedp, wc, w0)
    degs = _sc_hist(idx3, ones128, zeros1)

    ycat3 = ycat.reshape(80, 128, 64)
    degs4 = degs.reshape(2, 4, 80, 128)
    ysp3, yse3 = _tc_norm(ycat3, degs4)

    partp, parte = _sc_edges(ysp3.reshape(NP, 32), yse3.reshape(NP, 32),
                             idx3, zeros2)

    bsum = (b_post + b_emoji).reshape(1, 1, 32)
    score = _tc_final(partp.reshape(2, 80, 128, 32),
                      parte.reshape(2, 80, 128, 32),
                      degs4, bsum, pf, ss,
                      lin0_W[0:1, :], lin0_W[1:, :], lin0_b[None, :],
                      lin1_W, lin1_b[None, :])
    return score


# speedup vs baseline: 11.7807x; 1.4289x over previous
import functools

import jax
import jax.numpy as jnp
from jax import lax
from jax.experimental import pallas as pl
from jax.experimental.pallas import tpu as pltpu
from jax.experimental.pallas import tpu_sc as plsc

N = 10000
NP = 10240
E = 160000
EP = 163840
ROWS = EP // 128
NW = 32
RPW = ROWS // NW
NPT = NP // 16

f32 = jnp.float32
i32 = jnp.int32

_mesh = plsc.VectorSubcoreMesh(
    core_axis_name="c", subcore_axis_name="s", num_cores=2, num_subcores=16)

_sc_params = pltpu.CompilerParams(use_tc_tiling_on_sc=False)


@functools.partial(
    pl.kernel,
    out_type=jax.ShapeDtypeStruct((2, 4, NP), f32),
    mesh=_mesh,
    compiler_params=_sc_params,
    scratch_types=[
        pltpu.VMEM((RPW, 128), i32),
        pltpu.VMEM((RPW, 128), i32),
        pltpu.VMEM((RPW, 128), i32),
        pltpu.VMEM((RPW, 128), i32),
        pltpu.VMEM((128,), f32),
        pltpu.VMEM_SHARED((NP,), f32),
        pltpu.VMEM_SHARED((NP,), f32),
        pltpu.VMEM_SHARED((NP,), f32),
        pltpu.VMEM_SHARED((NP,), f32),
        pltpu.SemaphoreType.DMA,
    ])
def _sc_hist(idx_hbm, ones_hbm, zeros_hbm, out_hbm, iv0, iv1, iv2, iv3,
             ones_v, h0, h1, h2, h3, hsem):
    c = lax.axis_index("c")
    s = lax.axis_index("s")
    w = c * 16 + s
    hists = (h0, h1, h2, h3)
    idx_vs = (iv0, iv1, iv2, iv3)
    pltpu.sync_copy(ones_hbm, ones_v)
    for a in range(4):
        pltpu.sync_copy(zeros_hbm, hists[a].at[pl.ds(s * NPT, NPT)])
    for a in range(4):
        pltpu.sync_copy(idx_hbm.at[a].at[pl.ds(w * RPW, RPW)], idx_vs[a])
    plsc.subcore_barrier()

    def fire(j, carry):
        for a in range(4):
            pltpu.async_copy(ones_v, hists[a].at[idx_vs[a].at[j]], hsem,
                             add=True)
        return carry

    lax.fori_loop(0, RPW, fire, 0)

    def drain(j, carry):
        for a in range(4):
            pltpu.make_async_copy(ones_v, hists[a].at[idx_vs[a].at[j]],
                                  hsem).wait()
        return carry

    lax.fori_loop(0, RPW, drain, 0)
    plsc.subcore_barrier()
    for a in range(4):
        @pl.when(s == a)
        def _(a=a):
            pltpu.sync_copy(hists[a], out_hbm.at[c].at[a])


_NB = 8


@functools.partial(
    pl.kernel,
    out_type=(jax.ShapeDtypeStruct((2, NP, 32), f32),
              jax.ShapeDtypeStruct((2, NP, 32), f32)),
    mesh=_mesh,
    compiler_params=_sc_params,
    scratch_types=[
        pltpu.VMEM((RPW, 128), i32),
        pltpu.VMEM((RPW, 128), i32),
        pltpu.VMEM((_NB, 128, 32), f32),
        pltpu.VMEM_SHARED((NP, 32), f32),
        pltpu.VMEM_SHARED((NP, 32), f32),
        pltpu.VMEM_SHARED((NP, 32), f32),
        pltpu.VMEM_SHARED((NP, 32), f32),
    ] + [pltpu.SemaphoreType.DMA] * (2 * _NB))
def _sc_edges(ysp_hbm, yse_hbm, idx_hbm, zeros2_hbm, outp, oute,
              sidx, didx, rows, accp, acce, ysp_s, yse_s, *sems):
    gsem = sems[:_NB]
    ssem = sems[_NB:]
    c = lax.axis_index("c")
    s = lax.axis_index("s")
    w = c * 16 + s
    pltpu.sync_copy(zeros2_hbm, accp.at[pl.ds(s * NPT, NPT)])
    pltpu.sync_copy(zeros2_hbm, acce.at[pl.ds(s * NPT, NPT)])
    pltpu.sync_copy(ysp_hbm.at[pl.ds(s * NPT, NPT)],
                    ysp_s.at[pl.ds(s * NPT, NPT)])
    pltpu.sync_copy(yse_hbm.at[pl.ds(s * NPT, NPT)],
                    yse_s.at[pl.ds(s * NPT, NPT)])
    plsc.subcore_barrier()
    for ys, acc, a_src, a_dst in ((ysp_s, accp, 0, 1),
                                  (yse_s, acce, 2, 3)):
        pltpu.sync_copy(idx_hbm.at[a_src].at[pl.ds(w * RPW, RPW)], sidx)
        pltpu.sync_copy(idx_hbm.at[a_dst].at[pl.ds(w * RPW, RPW)], didx)

        def body(g, carry, ys=ys, acc=acc):
            gd = []
            for b in range(_NB):
                j = g * _NB + b

                @pl.when(g > 0)
                def _(b=b, j=j, acc=acc):
                    pltpu.make_async_copy(
                        rows.at[b], acc.at[didx.at[j - _NB]],
                        ssem[b]).wait()

                gd.append(pltpu.async_copy(ys.at[sidx.at[j]], rows.at[b],
                                           gsem[b]))
            for b in range(_NB):
                j = g * _NB + b
                gd[b].wait()
                pltpu.async_copy(rows.at[b], acc.at[didx.at[j]],
                                 ssem[b], add=True)
            return carry

        lax.fori_loop(0, RPW // _NB, body, 0)
        for b in range(_NB):
            pltpu.make_async_copy(rows.at[b],
                                  acc.at[didx.at[RPW - _NB + b]],
                                  ssem[b]).wait()
    plsc.subcore_barrier()
    pltpu.sync_copy(accp.at[pl.ds(s * NPT, NPT)],
                    outp.at[c].at[pl.ds(s * NPT, NPT)])
    pltpu.sync_copy(acce.at[pl.ds(s * NPT, NPT)],
                    oute.at[c].at[pl.ds(s * NPT, NPT)])


def _tc_proj_body(featb, seedb, wc, w0, ycat_o, pf_o, ss_o):
    i = pl.program_id(0)
    y = jnp.dot(featb[...], wc[...], preferred_element_type=f32)
    y = y + seedb[...] * w0[...]
    ycat_o[...] = y
    pfp = jnp.sum(featb[...], axis=0, keepdims=True)
    ssp = jnp.sum(seedb[...]).reshape(1, 1)

    @pl.when(i == 0)
    def _():
        pf_o[...] = pfp
        ss_o[...] = ssp

    @pl.when(i > 0)
    def _():
        pf_o[...] = pf_o[...] + pfp
        ss_o[...] = ss_o[...] + ssp


_BLK = 1024


def _tc_proj(featp, seedp, wc, w0):
    return pl.pallas_call(
        _tc_proj_body,
        grid=(NP // _BLK,),
        in_specs=[
            pl.BlockSpec((_BLK, 128), lambda i: (i, 0)),
            pl.BlockSpec((_BLK, 1), lambda i: (i, 0)),
            pl.BlockSpec((128, 64), lambda i: (0, 0)),
            pl.BlockSpec((1, 64), lambda i: (0, 0)),
        ],
        out_specs=[
            pl.BlockSpec((_BLK, 64), lambda i: (i, 0)),
            pl.BlockSpec((1, 128), lambda i: (0, 0)),
            pl.BlockSpec((1, 1), lambda i: (0, 0)),
        ],
        out_shape=[
            jax.ShapeDtypeStruct((NP, 64), f32),
            jax.ShapeDtypeStruct((1, 128), f32),
            jax.ShapeDtypeStruct((1, 1), f32),
        ],
    )(featp, seedp, wc, w0)


def _tc_norm_body(ycat, degs, ysp_o, yse_o):
    dsp = degs[0, 0] + degs[1, 0]
    dse = degs[0, 2] + degs[1, 2]
    nsp = jnp.where(dsp > 0, lax.rsqrt(dsp), 0.0)[..., None]
    nse = jnp.where(dse > 0, lax.rsqrt(dse), 0.0)[..., None]
    yc = ycat[...]
    ysp_o[...] = yc[:, :, :32] * nsp
    yse_o[...] = yc[:, :, 32:] * nse


def _tc_norm(ycat3, degs4):
    return pl.pallas_call(
        _tc_norm_body,
        out_shape=[
            jax.ShapeDtypeStruct((80, 128, 32), f32),
            jax.ShapeDtypeStruct((80, 128, 32), f32),
        ],
    )(ycat3, degs4)


def _tc_final_body(partp, parte, degs, bsum, pf, ss, l0w0, l0wf, l0b,
                   l1w, l1b, out):
    aggp = partp[0] + partp[1]
    agge = parte[0] + parte[1]
    ddp = degs[0, 1] + degs[1, 1]
    dde = degs[0, 3] + degs[1, 3]
    np_ = jnp.where(ddp > 0, lax.rsqrt(ddp), 0.0)[..., None]
    ne_ = jnp.where(dde > 0, lax.rsqrt(dde), 0.0)[..., None]
    h = jnp.maximum(aggp * np_ + agge * ne_ + bsum[...], 0.0)
    row = (lax.broadcasted_iota(i32, (80, 128, 1), 0) * 128
           + lax.broadcasted_iota(i32, (80, 128, 1), 1))
    h = jnp.where(row < N, h, 0.0)
    pooled1 = h.sum(axis=1).sum(axis=0)[None, :]
    score = (ss[...] * l0w0[...]
             + jnp.dot(pf[...], l0wf[...], preferred_element_type=f32)
             + l0b[...]
             + jnp.dot(pooled1, l1w[...], preferred_element_type=f32)
             + l1b[...])
    out[...] = score


def _tc_final(partp, parte, degs4, bsum, pf, ss, l0w0, l0wf, l0b, l1w, l1b):
    return pl.pallas_call(
        _tc_final_body,
        out_shape=jax.ShapeDtypeStruct((1, 32), f32),
    )(partp, parte, degs4, bsum, pf, ss, l0w0, l0wf, l0b, l1w, l1b)


def kernel(feat, seed, edge_index_post, edge_index_emoji, W_post, b_post,
           W_emoji, b_emoji, lin0_W, lin0_b, lin1_W, lin1_b):
    featp = jnp.zeros((NP, 128), f32).at[:N].set(feat)
    seedp = jnp.zeros((NP, 1), f32).at[:N, 0].set(seed.astype(f32))
    idx = jnp.full((4, EP), NP - 1, i32)
    idx = idx.at[0, :E].set(edge_index_post[0].astype(i32))
    idx = idx.at[1, :E].set(edge_index_post[1].astype(i32))
    idx = idx.at[2, :E].set(edge_index_emoji[0].astype(i32))
    idx = idx.at[3, :E].set(edge_index_emoji[1].astype(i32))
    idx3 = idx.reshape(4, ROWS, 128)
    wc = jnp.concatenate([W_post[1:], W_emoji[1:]], axis=1)
    w0 = jnp.concatenate([W_post[0], W_emoji[0]])[None, :]
    ones128 = jnp.ones((128,), f32)
    zeros1 = jnp.zeros((NPT,), f32)
    zeros2 = jnp.zeros((NPT, 32), f32)

    ycat, pf, ss = _tc_proj(featp, seedp, wc, w0)
    degs = _sc_hist(idx3, ones128, zeros1)

    ycat3 = ycat.reshape(80, 128, 64)
    degs4 = degs.reshape(2, 4, 80, 128)
    ysp3, yse3 = _tc_norm(ycat3, degs4)

    partp, parte = _sc_edges(ysp3.reshape(NP, 32), yse3.reshape(NP, 32),
                             idx3, zeros2)

    bsum = (b_post + b_emoji).reshape(1, 1, 32)
    score = _tc_final(partp.reshape(2, 80, 128, 32),
                      parte.reshape(2, 80, 128, 32),
                      degs4, bsum, pf, ss,
                      lin0_W[0:1, :], lin0_W[1:, :], lin0_b[None, :],
                      lin1_W, lin1_b[None, :])
    return score
